# Initial kernel scaffold; baseline (speedup 1.0000x reference)
import jax
import jax.numpy as jnp
from jax.experimental import pallas as pl


def kernel(x, edge_index1, edge_index2, emb0, emb1, emb2, emb3, W_in, b_in, W_out, b_out):
    raise NotImplementedError("write your pallas kernel here")


# speedup vs baseline: 5.8719x; 5.8719x over previous
import functools

import jax
import jax.numpy as jnp
from jax import lax
from jax.experimental import pallas as pl
from jax.experimental.pallas import tpu as pltpu
from jax.experimental.pallas import tpu_sc as plsc

N_NODES = 100000
N_EDGES = 1600000

NC = 2
NS = 16
L = 16

GATHER_ROWS_PAD = 409600
GKB = GATHER_ROWS_PAD // (NC * NS * 128)


def _gather_body(tbl_hbm, idx_hbm, out_hbm, idx_v, data_v, sem):
    c = lax.axis_index("c")
    s = lax.axis_index("s")
    wid = s * NC + c
    pltpu.sync_copy(idx_hbm.at[wid], idx_v)

    def chunk(j, carry):
        pltpu.async_copy(tbl_hbm.at[idx_v.at[j]], data_v, sem).wait()
        base = wid * (GKB * 128) + j * 128
        pltpu.sync_copy(data_v, out_hbm.at[pl.ds(base, 128)])
        return carry

    lax.fori_loop(0, GKB, chunk, 0)


_gather_call = pl.kernel(
    _gather_body,
    out_type=jax.ShapeDtypeStruct((GATHER_ROWS_PAD, 16), jnp.float32),
    compiler_params=pltpu.CompilerParams(use_tc_tiling_on_sc=False),
    mesh=plsc.VectorSubcoreMesh(
        core_axis_name="c", subcore_axis_name="s", num_cores=NC,
        num_subcores=NS),
    scratch_types=[
        pltpu.VMEM((GKB, 128), jnp.int32),
        pltpu.VMEM((128, 16), jnp.float32),
        pltpu.SemaphoreType.DMA,
    ],
)

EB = 196
EJ = 4
E_PAD = NS * EB * EJ * 128
ACC_ROWS = 100096
ACC_SLICE = ACC_ROWS // NS


def _seg_body(h2_hbm, src_hbm, dst_hbm, zacc_hbm, zdeg_hbm,
              sums_hbm, deg_hbm,
              src_v, dst_v, gidx_v, d0, d1, d2, d3, ones_v,
              acc_sh, deg_sh, sem):
    c = lax.axis_index("c")
    s = lax.axis_index("s")
    data = (d0, d1, d2, d3)

    pltpu.sync_copy(zacc_hbm.at[pl.ds(s * ACC_SLICE, ACC_SLICE)],
                    acc_sh.at[pl.ds(s * ACC_SLICE, ACC_SLICE)])

    @pl.when(c == 0)
    def _():
        pltpu.sync_copy(zdeg_hbm.at[pl.ds(s * ACC_SLICE, ACC_SLICE)],
                        deg_sh.at[pl.ds(s * ACC_SLICE, ACC_SLICE)])
        for t in range(128 // L):
            ones_v[pl.ds(t * L, L)] = jnp.ones((L,), jnp.float32)

    plsc.subcore_barrier()

    def block(b, carry):
        pltpu.sync_copy(src_hbm.at[s, b], src_v)
        pltpu.sync_copy(dst_hbm.at[s, b], dst_v)
        for j in range(EJ):
            for t in range(128 // L):
                sv = src_v[j, pl.ds(t * L, L)]
                gidx_v[j, pl.ds(t * L, L)] = sv * 2 + c
        for j in range(EJ):
            pltpu.async_copy(h2_hbm.at[gidx_v.at[j]], data[j], sem).wait()
        for j in range(EJ):
            pltpu.sync_copy(data[j], acc_sh.at[dst_v.at[j]], add=True)

        @pl.when(c == 0)
        def _():
            for j in range(EJ):
                pltpu.sync_copy(ones_v, deg_sh.at[dst_v.at[j]], add=True)

        return carry

    lax.fori_loop(0, EB, block, 0)
    plsc.subcore_barrier()

    pltpu.sync_copy(acc_sh.at[pl.ds(s * ACC_SLICE, ACC_SLICE)],
                    sums_hbm.at[c, pl.ds(s * ACC_SLICE, ACC_SLICE)])

    @pl.when(c == 0)
    def _():
        pltpu.sync_copy(deg_sh.at[pl.ds(s * ACC_SLICE, ACC_SLICE)],
                        deg_hbm.at[pl.ds(s * ACC_SLICE, ACC_SLICE)])


_seg_call = pl.kernel(
    _seg_body,
    out_type=(
        jax.ShapeDtypeStruct((NC, ACC_ROWS, 16), jnp.float32),
        jax.ShapeDtypeStruct((ACC_ROWS,), jnp.float32),
    ),
    compiler_params=pltpu.CompilerParams(use_tc_tiling_on_sc=False),
    mesh=plsc.VectorSubcoreMesh(
        core_axis_name="c", subcore_axis_name="s", num_cores=NC,
        num_subcores=NS),
    scratch_types=[
        pltpu.VMEM((EJ, 128), jnp.int32),
        pltpu.VMEM((EJ, 128), jnp.int32),
        pltpu.VMEM((EJ, 128), jnp.int32),
        pltpu.VMEM((128, 16), jnp.float32),
        pltpu.VMEM((128, 16), jnp.float32),
        pltpu.VMEM((128, 16), jnp.float32),
        pltpu.VMEM((128, 16), jnp.float32),
        pltpu.VMEM((128,), jnp.float32),
        pltpu.VMEM_SHARED((ACC_ROWS, 16), jnp.float32),
        pltpu.VMEM_SHARED((ACC_ROWS,), jnp.float32),
        pltpu.SemaphoreType.DMA,
    ],
)


BLK = 2000


def _tc_in_body(x_ref, w_ref, b_ref, o_ref):
    o_ref[...] = (
        jnp.dot(x_ref[...], w_ref[...], preferred_element_type=jnp.float32)
        + b_ref[...])


def _tc_mid_body(h_ref, s0_ref, s1_ref, deg_ref, o_ref):
    r = 1.0 / jnp.maximum(deg_ref[...], 1.0)
    mean = jnp.concatenate([s0_ref[...], s1_ref[...]], axis=1) * r
    o_ref[...] = jnp.maximum(h_ref[...] + mean, 0.0)


def _tc_out_body(h_ref, s0_ref, s1_ref, deg_ref, w_ref, b_ref, o_ref):
    r = 1.0 / jnp.maximum(deg_ref[...], 1.0)
    mean = jnp.concatenate([s0_ref[...], s1_ref[...]], axis=1) * r
    h = h_ref[...] + mean
    o_ref[...] = (
        jnp.dot(h, w_ref[...], preferred_element_type=jnp.float32)
        + b_ref[...])


def _row_spec(cols):
    return pl.BlockSpec((BLK, cols), lambda i: (i, 0))


def _full_spec(shape):
    return pl.BlockSpec(shape, lambda i: tuple(0 for _ in shape))


_tc_in = pl.pallas_call(
    _tc_in_body,
    grid=(N_NODES // BLK,),
    in_specs=[_row_spec(64), _full_spec((64, 32)), _full_spec((1, 32))],
    out_specs=_row_spec(32),
    out_shape=jax.ShapeDtypeStruct((N_NODES, 32), jnp.float32),
)

_tc_mid = pl.pallas_call(
    _tc_mid_body,
    grid=(N_NODES // BLK,),
    in_specs=[_row_spec(32), _row_spec(16), _row_spec(16), _row_spec(1)],
    out_specs=_row_spec(32),
    out_shape=jax.ShapeDtypeStruct((N_NODES, 32), jnp.float32),
)

_tc_out = pl.pallas_call(
    _tc_out_body,
    grid=(N_NODES // BLK,),
    in_specs=[_row_spec(32), _row_spec(16), _row_spec(16), _row_spec(1),
              _full_spec((32, 32)), _full_spec((1, 32))],
    out_specs=_row_spec(32),
    out_shape=jax.ShapeDtypeStruct((N_NODES, 32), jnp.float32),
)


def _prep_edges(edge_index):
    src = edge_index[0].astype(jnp.int32)
    dst = edge_index[1].astype(jnp.int32)
    pad = E_PAD - N_EDGES
    src = jnp.concatenate([src, jnp.zeros((pad,), jnp.int32)])
    dst = jnp.concatenate([dst, jnp.full((pad,), N_NODES, jnp.int32)])
    return (src.reshape(NS, EB, EJ, 128), dst.reshape(NS, EB, EJ, 128))


def _mean_inputs(h, edges):
    h2 = h.reshape(2 * N_NODES, 16)
    zacc = jnp.zeros((ACC_ROWS, 16), jnp.float32)
    zdeg = jnp.zeros((ACC_ROWS,), jnp.float32)
    sums, deg = _seg_call(h2, edges[0], edges[1], zacc, zdeg)
    s0 = sums[0, :N_NODES, :]
    s1 = sums[1, :N_NODES, :]
    return s0, s1, deg[:N_NODES].reshape(N_NODES, 1)


@jax.jit
def kernel(x, edge_index1, edge_index2, emb0, emb1, emb2, emb3,
           W_in, b_in, W_out, b_out):
    tbl = jnp.concatenate([emb0, emb1, emb2, emb3], axis=0)
    offs = jnp.array([0, 1000, 2000, 2100], jnp.int32)
    gidx = (x.astype(jnp.int32) + offs).reshape(-1)
    gidx = jnp.concatenate(
        [gidx, jnp.zeros((GATHER_ROWS_PAD - 4 * N_NODES,), jnp.int32)])
    feats_flat = _gather_call(tbl, gidx.reshape(NC * NS, GKB, 128))
    feats = feats_flat[:4 * N_NODES].reshape(N_NODES, 64)

    h = _tc_in(feats, W_in, b_in.reshape(1, 32))

    e1 = _prep_edges(edge_index1)
    s0, s1, deg = _mean_inputs(h, e1)
    h = _tc_mid(h, s0, s1, deg)

    e2 = _prep_edges(edge_index2)
    s0, s1, deg = _mean_inputs(h, e2)
    return _tc_out(h, s0, s1, deg, W_out, b_out.reshape(1, 32))


# speedup vs baseline: 9.1829x; 1.5639x over previous
import jax
import jax.numpy as jnp
from jax import lax
from jax.experimental import pallas as pl
from jax.experimental.pallas import tpu as pltpu
from jax.experimental.pallas import tpu_sc as plsc

N_NODES = 100000
N_EDGES = 1600000

NC = 2
NS = 16
L = 16

GATHER_ROWS_PAD = 409600
GKB = GATHER_ROWS_PAD // (NC * NS * 128)
GG = 4


def _gather_body(tbl_hbm, idx_hbm, out_hbm, idx_v, d0, d1, d2, d3,
                 gsem, osem):
    c = lax.axis_index("c")
    s = lax.axis_index("s")
    wid = s * NC + c
    data = (d0, d1, d2, d3)
    pltpu.sync_copy(idx_hbm.at[wid], idx_v)

    def out_ref(g, j):
        base = wid * (GKB * 128) + (g * GG + j) * 128
        return out_hbm.at[pl.ds(base, 128)]

    def group(g, carry):
        @pl.when(g > 0)
        def _():
            for j in range(GG):
                pltpu.make_async_copy(data[j], out_ref(g - 1, j),
                                      osem).wait()
        descs = [
            pltpu.async_copy(tbl_hbm.at[idx_v.at[g * GG + j]], data[j],
                             gsem)
            for j in range(GG)]
        for j in range(GG):
            descs[j].wait()
        for j in range(GG):
            pltpu.async_copy(data[j], out_ref(g, j), osem)
        return carry

    lax.fori_loop(0, GKB // GG, group, 0)
    for j in range(GG):
        pltpu.make_async_copy(data[j], out_ref(GKB // GG - 1, j),
                              osem).wait()


_gather_call = pl.kernel(
    _gather_body,
    out_type=jax.ShapeDtypeStruct((GATHER_ROWS_PAD, 16), jnp.float32),
    compiler_params=pltpu.CompilerParams(use_tc_tiling_on_sc=False),
    mesh=plsc.VectorSubcoreMesh(
        core_axis_name="c", subcore_axis_name="s", num_cores=NC,
        num_subcores=NS),
    scratch_types=[
        pltpu.VMEM((GKB, 128), jnp.int32),
        pltpu.VMEM((128, 16), jnp.float32),
        pltpu.VMEM((128, 16), jnp.float32),
        pltpu.VMEM((128, 16), jnp.float32),
        pltpu.VMEM((128, 16), jnp.float32),
        pltpu.SemaphoreType.DMA,
        pltpu.SemaphoreType.DMA,
    ],
)

EB = 196
EJ = 4
E_PAD = NS * EB * EJ * 128
ACC_ROWS = 100096
ACC_SLICE = ACC_ROWS // NS


def _seg_body(h2_hbm, idx_hbm, zacc_hbm, zdeg_hbm,
              sums_hbm, deg_hbm,
              ia, ib, ga, gb, a0, a1, a2, a3, b0, b1, b2, b3, ones_v,
              acc_sh, deg_sh, gsem, ssa, ssb):
    c = lax.axis_index("c")
    s = lax.axis_index("s")
    slots = ((ia, ga, (a0, a1, a2, a3), ssa),
             (ib, gb, (b0, b1, b2, b3), ssb))

    pltpu.sync_copy(zacc_hbm.at[pl.ds(s * ACC_SLICE, ACC_SLICE)],
                    acc_sh.at[pl.ds(s * ACC_SLICE, ACC_SLICE)])
    pltpu.sync_copy(zdeg_hbm.at[pl.ds(s * ACC_SLICE, ACC_SLICE)],
                    deg_sh.at[pl.ds(s * ACC_SLICE, ACC_SLICE)])
    for t in range(128 // L):
        ones_v[pl.ds(t * L, L)] = jnp.ones((L,), jnp.float32)
    plsc.subcore_barrier()

    def fire_scatters(idx_v, data, ssem):
        for j in range(EJ):
            pltpu.async_copy(data[j], acc_sh.at[idx_v.at[1, j]], ssem,
                             add=True)
        @pl.when(c == 0)
        def _():
            for j in range(EJ // 2):
                pltpu.async_copy(ones_v, deg_sh.at[idx_v.at[1, j]], ssem,
                                 add=True)

        @pl.when(c == 1)
        def _():
            for j in range(EJ // 2, EJ):
                pltpu.async_copy(ones_v, deg_sh.at[idx_v.at[1, j]], ssem,
                                 add=True)

    def drain_scatters(idx_v, data, ssem):
        for j in range(EJ):
            pltpu.make_async_copy(data[j], acc_sh.at[idx_v.at[1, j]],
                                  ssem).wait()

        @pl.when(c == 0)
        def _():
            for j in range(EJ // 2):
                pltpu.make_async_copy(ones_v, deg_sh.at[idx_v.at[1, j]],
                                      ssem).wait()

        @pl.when(c == 1)
        def _():
            for j in range(EJ // 2, EJ):
                pltpu.make_async_copy(ones_v, deg_sh.at[idx_v.at[1, j]],
                                      ssem).wait()

    def half_step(i, slot_id, block):
        idx_v, gidx_v, data, ssem = slots[slot_id]
        @pl.when(i > 0)
        def _():
            drain_scatters(idx_v, data, ssem)
        pltpu.sync_copy(idx_hbm.at[s, block], idx_v)
        for j in range(EJ):
            for t in range(128 // L):
                sv = idx_v[0, j, pl.ds(t * L, L)]
                gidx_v[j, pl.ds(t * L, L)] = sv * 2 + c
        descs = [
            pltpu.async_copy(h2_hbm.at[gidx_v.at[j]], data[j], gsem)
            for j in range(EJ)]
        for j in range(EJ):
            descs[j].wait()
        fire_scatters(idx_v, data, ssem)

    def pair(i, carry):
        half_step(i, 0, 2 * i)
        half_step(i, 1, 2 * i + 1)
        return carry

    lax.fori_loop(0, EB // 2, pair, 0)
    for idx_v, _g, data, ssem in slots:
        drain_scatters(idx_v, data, ssem)
    plsc.subcore_barrier()

    pltpu.sync_copy(acc_sh.at[pl.ds(s * ACC_SLICE, ACC_SLICE)],
                    sums_hbm.at[c, pl.ds(s * ACC_SLICE, ACC_SLICE)])
    pltpu.sync_copy(deg_sh.at[pl.ds(s * ACC_SLICE, ACC_SLICE)],
                    deg_hbm.at[c, pl.ds(s * ACC_SLICE, ACC_SLICE)])


_seg_call = pl.kernel(
    _seg_body,
    out_type=(
        jax.ShapeDtypeStruct((NC, ACC_ROWS, 16), jnp.float32),
        jax.ShapeDtypeStruct((NC, ACC_ROWS), jnp.float32),
    ),
    compiler_params=pltpu.CompilerParams(use_tc_tiling_on_sc=False),
    mesh=plsc.VectorSubcoreMesh(
        core_axis_name="c", subcore_axis_name="s", num_cores=NC,
        num_subcores=NS),
    scratch_types=[
        pltpu.VMEM((2, EJ, 128), jnp.int32),
        pltpu.VMEM((2, EJ, 128), jnp.int32),
        pltpu.VMEM((EJ, 128), jnp.int32),
        pltpu.VMEM((EJ, 128), jnp.int32),
        pltpu.VMEM((128, 16), jnp.float32),
        pltpu.VMEM((128, 16), jnp.float32),
        pltpu.VMEM((128, 16), jnp.float32),
        pltpu.VMEM((128, 16), jnp.float32),
        pltpu.VMEM((128, 16), jnp.float32),
        pltpu.VMEM((128, 16), jnp.float32),
        pltpu.VMEM((128, 16), jnp.float32),
        pltpu.VMEM((128, 16), jnp.float32),
        pltpu.VMEM((128,), jnp.float32),
        pltpu.VMEM_SHARED((ACC_ROWS, 16), jnp.float32),
        pltpu.VMEM_SHARED((ACC_ROWS,), jnp.float32),
        pltpu.SemaphoreType.DMA,
        pltpu.SemaphoreType.DMA,
        pltpu.SemaphoreType.DMA,
    ],
)


BLK = 2000


def _tc_in_body(x_ref, w_ref, b_ref, o_ref):
    o_ref[...] = (
        jnp.dot(x_ref[...], w_ref[...], preferred_element_type=jnp.float32)
        + b_ref[...])


def _tc_mid_body(h_ref, s0_ref, s1_ref, d0_ref, d1_ref, o_ref):
    r = 1.0 / jnp.maximum(d0_ref[...] + d1_ref[...], 1.0)
    mean = jnp.concatenate([s0_ref[...], s1_ref[...]], axis=1) * r
    o_ref[...] = jnp.maximum(h_ref[...] + mean, 0.0)


def _tc_out_body(h_ref, s0_ref, s1_ref, d0_ref, d1_ref, w_ref, b_ref,
                 o_ref):
    r = 1.0 / jnp.maximum(d0_ref[...] + d1_ref[...], 1.0)
    mean = jnp.concatenate([s0_ref[...], s1_ref[...]], axis=1) * r
    h = h_ref[...] + mean
    o_ref[...] = (
        jnp.dot(h, w_ref[...], preferred_element_type=jnp.float32)
        + b_ref[...])


def _row_spec(cols):
    return pl.BlockSpec((BLK, cols), lambda i: (i, 0))


def _full_spec(shape):
    return pl.BlockSpec(shape, lambda i: tuple(0 for _ in shape))


_tc_in = pl.pallas_call(
    _tc_in_body,
    grid=(N_NODES // BLK,),
    in_specs=[_row_spec(64), _full_spec((64, 32)), _full_spec((1, 32))],
    out_specs=_row_spec(32),
    out_shape=jax.ShapeDtypeStruct((N_NODES, 32), jnp.float32),
)

_tc_mid = pl.pallas_call(
    _tc_mid_body,
    grid=(N_NODES // BLK,),
    in_specs=[_row_spec(32), _row_spec(16), _row_spec(16), _row_spec(1),
              _row_spec(1)],
    out_specs=_row_spec(32),
    out_shape=jax.ShapeDtypeStruct((N_NODES, 32), jnp.float32),
)

_tc_out = pl.pallas_call(
    _tc_out_body,
    grid=(N_NODES // BLK,),
    in_specs=[_row_spec(32), _row_spec(16), _row_spec(16), _row_spec(1),
              _row_spec(1), _full_spec((32, 32)), _full_spec((1, 32))],
    out_specs=_row_spec(32),
    out_shape=jax.ShapeDtypeStruct((N_NODES, 32), jnp.float32),
)


def _prep_edges(edge_index):
    src = edge_index[0].astype(jnp.int32)
    dst = edge_index[1].astype(jnp.int32)
    pad = E_PAD - N_EDGES
    src = jnp.concatenate([src, jnp.zeros((pad,), jnp.int32)])
    dst = jnp.concatenate([dst, jnp.full((pad,), N_NODES, jnp.int32)])
    return jnp.stack([src.reshape(NS, EB, EJ, 128),
                      dst.reshape(NS, EB, EJ, 128)], axis=2)


def _mean_inputs(h, edges):
    h2 = h.reshape(2 * N_NODES, 16)
    zacc = jnp.zeros((ACC_ROWS, 16), jnp.float32)
    zdeg = jnp.zeros((ACC_ROWS,), jnp.float32)
    sums, deg = _seg_call(h2, edges, zacc, zdeg)
    s0 = sums[0, :N_NODES, :]
    s1 = sums[1, :N_NODES, :]
    d0 = deg[0, :N_NODES].reshape(N_NODES, 1)
    d1 = deg[1, :N_NODES].reshape(N_NODES, 1)
    return s0, s1, d0, d1


@jax.jit
def kernel(x, edge_index1, edge_index2, emb0, emb1, emb2, emb3,
           W_in, b_in, W_out, b_out):
    tbl = jnp.concatenate([emb0, emb1, emb2, emb3], axis=0)
    offs = jnp.array([0, 1000, 2000, 2100], jnp.int32)
    gidx = (x.astype(jnp.int32) + offs).reshape(-1)
    gidx = jnp.concatenate(
        [gidx, jnp.zeros((GATHER_ROWS_PAD - 4 * N_NODES,), jnp.int32)])
    feats_flat = _gather_call(tbl, gidx.reshape(NC * NS, GKB, 128))
    feats = feats_flat[:4 * N_NODES].reshape(N_NODES, 64)

    h = _tc_in(feats, W_in, b_in.reshape(1, 32))

    e1 = _prep_edges(edge_index1)
    s0, s1, d0, d1 = _mean_inputs(h, e1)
    h = _tc_mid(h, s0, s1, d0, d1)

    e2 = _prep_edges(edge_index2)
    s0, s1, d0, d1 = _mean_inputs(h, e2)
    return _tc_out(h, s0, s1, d0, d1, W_out, b_out.reshape(1, 32))


# speedup vs baseline: 10.0059x; 1.0896x over previous
import jax
import jax.numpy as jnp
from jax import lax
from jax.experimental import pallas as pl
from jax.experimental.pallas import tpu as pltpu
from jax.experimental.pallas import tpu_sc as plsc

N_NODES = 100000
N_EDGES = 1600000

NC = 2
NS = 16
L = 16

GATHER_ROWS_PAD = 409600
GKB = GATHER_ROWS_PAD // (NC * NS * 128)
GG = 4


def _gather_body(tbl_hbm, idx_hbm, out_hbm, idx_v, d0, d1, d2, d3,
                 gsem, osem):
    c = lax.axis_index("c")
    s = lax.axis_index("s")
    wid = s * NC + c
    data = (d0, d1, d2, d3)
    pltpu.sync_copy(idx_hbm.at[wid], idx_v)

    def out_ref(g, j):
        base = wid * (GKB * 128) + (g * GG + j) * 128
        return out_hbm.at[pl.ds(base, 128)]

    def group(g, carry):
        @pl.when(g > 0)
        def _():
            for j in range(GG):
                pltpu.make_async_copy(data[j], out_ref(g - 1, j),
                                      osem).wait()
        descs = [
            pltpu.async_copy(tbl_hbm.at[idx_v.at[g * GG + j]], data[j],
                             gsem)
            for j in range(GG)]
        for j in range(GG):
            descs[j].wait()
        for j in range(GG):
            pltpu.async_copy(data[j], out_ref(g, j), osem)
        return carry

    lax.fori_loop(0, GKB // GG, group, 0)
    for j in range(GG):
        pltpu.make_async_copy(data[j], out_ref(GKB // GG - 1, j),
                              osem).wait()


_gather_call = pl.kernel(
    _gather_body,
    out_type=jax.ShapeDtypeStruct((GATHER_ROWS_PAD, 16), jnp.float32),
    compiler_params=pltpu.CompilerParams(use_tc_tiling_on_sc=False),
    mesh=plsc.VectorSubcoreMesh(
        core_axis_name="c", subcore_axis_name="s", num_cores=NC,
        num_subcores=NS),
    scratch_types=[
        pltpu.VMEM((GKB, 128), jnp.int32),
        pltpu.VMEM((128, 16), jnp.float32),
        pltpu.VMEM((128, 16), jnp.float32),
        pltpu.VMEM((128, 16), jnp.float32),
        pltpu.VMEM((128, 16), jnp.float32),
        pltpu.SemaphoreType.DMA,
        pltpu.SemaphoreType.DMA,
    ],
)

EB = 196
EJ = 4
E_PAD = NS * EB * EJ * 128
ACC_ROWS = 100096
ACC_SLICE = ACC_ROWS // NS


def _seg_body(h2_hbm, idx_hbm, zacc_hbm, zdeg_hbm,
              sums_hbm, deg_hbm,
              ia, ib, ga, gb, a0, a1, a2, a3, b0, b1, b2, b3, ones_v,
              acc_sh, deg_sh, gsem, ssa, ssb):
    c = lax.axis_index("c")
    s = lax.axis_index("s")
    slots = ((ia, ga, (a0, a1, a2, a3), ssa),
             (ib, gb, (b0, b1, b2, b3), ssb))

    pltpu.sync_copy(zacc_hbm.at[pl.ds(s * ACC_SLICE, ACC_SLICE)],
                    acc_sh.at[pl.ds(s * ACC_SLICE, ACC_SLICE)])
    pltpu.sync_copy(zdeg_hbm.at[pl.ds(s * ACC_SLICE, ACC_SLICE)],
                    deg_sh.at[pl.ds(s * ACC_SLICE, ACC_SLICE)])
    for t in range(128 // L):
        ones_v[pl.ds(t * L, L)] = jnp.ones((L,), jnp.float32)
    plsc.subcore_barrier()

    def fire_scatters(idx_v, data, ssem):
        for j in range(EJ):
            pltpu.async_copy(data[j], acc_sh.at[idx_v.at[1, j]], ssem,
                             add=True)
        @pl.when(c == 0)
        def _():
            for j in range(EJ // 2):
                pltpu.async_copy(ones_v, deg_sh.at[idx_v.at[1, j]], ssem,
                                 add=True)

        @pl.when(c == 1)
        def _():
            for j in range(EJ // 2, EJ):
                pltpu.async_copy(ones_v, deg_sh.at[idx_v.at[1, j]], ssem,
                                 add=True)

    def drain_scatters(idx_v, data, ssem):
        for j in range(EJ):
            pltpu.make_async_copy(data[j], acc_sh.at[idx_v.at[1, j]],
                                  ssem).wait()

        @pl.when(c == 0)
        def _():
            for j in range(EJ // 2):
                pltpu.make_async_copy(ones_v, deg_sh.at[idx_v.at[1, j]],
                                      ssem).wait()

        @pl.when(c == 1)
        def _():
            for j in range(EJ // 2, EJ):
                pltpu.make_async_copy(ones_v, deg_sh.at[idx_v.at[1, j]],
                                      ssem).wait()

    def half_step(i, slot_id, block):
        idx_v, gidx_v, data, ssem = slots[slot_id]
        @pl.when(i > 0)
        def _():
            drain_scatters(idx_v, data, ssem)
        pltpu.sync_copy(idx_hbm.at[s, block], idx_v)
        for j in range(EJ):
            for t in range(128 // L):
                sv = idx_v[0, j, pl.ds(t * L, L)]
                gidx_v[j, pl.ds(t * L, L)] = sv * 2 + c
        descs = [
            pltpu.async_copy(h2_hbm.at[gidx_v.at[j]], data[j], gsem)
            for j in range(EJ)]
        for j in range(EJ):
            descs[j].wait()
        fire_scatters(idx_v, data, ssem)

    def pair(i, carry):
        half_step(i, 0, 2 * i)
        half_step(i, 1, 2 * i + 1)
        return carry

    lax.fori_loop(0, EB // 2, pair, 0)
    for idx_v, _g, data, ssem in slots:
        drain_scatters(idx_v, data, ssem)
    plsc.subcore_barrier()

    pltpu.sync_copy(acc_sh.at[pl.ds(s * ACC_SLICE, ACC_SLICE)],
                    sums_hbm.at[c, pl.ds(s * ACC_SLICE, ACC_SLICE)])
    pltpu.sync_copy(deg_sh.at[pl.ds(s * ACC_SLICE, ACC_SLICE)],
                    deg_hbm.at[c, pl.ds(s * ACC_SLICE, ACC_SLICE)])


_seg_call = pl.kernel(
    _seg_body,
    out_type=(
        jax.ShapeDtypeStruct((NC, ACC_ROWS, 16), jnp.float32),
        jax.ShapeDtypeStruct((NC, ACC_ROWS), jnp.float32),
    ),
    compiler_params=pltpu.CompilerParams(use_tc_tiling_on_sc=False),
    mesh=plsc.VectorSubcoreMesh(
        core_axis_name="c", subcore_axis_name="s", num_cores=NC,
        num_subcores=NS),
    scratch_types=[
        pltpu.VMEM((2, EJ, 128), jnp.int32),
        pltpu.VMEM((2, EJ, 128), jnp.int32),
        pltpu.VMEM((EJ, 128), jnp.int32),
        pltpu.VMEM((EJ, 128), jnp.int32),
        pltpu.VMEM((128, 16), jnp.float32),
        pltpu.VMEM((128, 16), jnp.float32),
        pltpu.VMEM((128, 16), jnp.float32),
        pltpu.VMEM((128, 16), jnp.float32),
        pltpu.VMEM((128, 16), jnp.float32),
        pltpu.VMEM((128, 16), jnp.float32),
        pltpu.VMEM((128, 16), jnp.float32),
        pltpu.VMEM((128, 16), jnp.float32),
        pltpu.VMEM((128,), jnp.float32),
        pltpu.VMEM_SHARED((ACC_ROWS, 16), jnp.float32),
        pltpu.VMEM_SHARED((ACC_ROWS,), jnp.float32),
        pltpu.SemaphoreType.DMA,
        pltpu.SemaphoreType.DMA,
        pltpu.SemaphoreType.DMA,
    ],
)


BLK = 2000


def _interleave(s0, s1):
    return jnp.stack([s0, s1], axis=1).reshape(2 * BLK, 16)


def _deinterleave(h16):
    h3 = h16.reshape(BLK, 2, 16)
    return jnp.concatenate([h3[:, 0, :], h3[:, 1, :]], axis=1)


def _tc_in_body(x0_ref, x1_ref, x2_ref, x3_ref, w_ref, b_ref, o_ref):
    xs = (x0_ref, x1_ref, x2_ref, x3_ref)
    h = b_ref[...]
    for f in range(4):
        h = h + jnp.dot(xs[f][...], w_ref[f * 16:(f + 1) * 16, :],
                        preferred_element_type=jnp.float32)
    o_ref[...] = _interleave(h[:, :16], h[:, 16:])


def _tc_mid_body(h_ref, s0_ref, s1_ref, d_ref, o_ref):
    r = 1.0 / jnp.maximum(d_ref[...], 1.0)
    mean16 = _interleave(s0_ref[0] * r, s1_ref[0] * r)
    o_ref[...] = jnp.maximum(h_ref[...] + mean16, 0.0)


def _tc_out_body(h_ref, s0_ref, s1_ref, d_ref, w_ref, b_ref, o_ref):
    r = 1.0 / jnp.maximum(d_ref[...], 1.0)
    mean16 = _interleave(s0_ref[0] * r, s1_ref[0] * r)
    h = _deinterleave(h_ref[...] + mean16)
    o_ref[...] = (
        jnp.dot(h, w_ref[...], preferred_element_type=jnp.float32)
        + b_ref[...])


def _full_spec(shape):
    return pl.BlockSpec(shape, lambda i: tuple(0 for _ in shape))


def _h16_spec():
    return pl.BlockSpec((2 * BLK, 16), lambda i: (i, 0))


def _sum_spec(core):
    return pl.BlockSpec((1, BLK, 16), lambda i, core=core: (core, i, 0))


_tc_in = pl.pallas_call(
    _tc_in_body,
    grid=(N_NODES // BLK,),
    in_specs=[pl.BlockSpec((BLK, 16), lambda i, f=f: (f * (N_NODES // BLK)
                                                      + i, 0))
              for f in range(4)]
    + [_full_spec((64, 32)), _full_spec((1, 32))],
    out_specs=_h16_spec(),
    out_shape=jax.ShapeDtypeStruct((2 * N_NODES, 16), jnp.float32),
)

_tc_mid = pl.pallas_call(
    _tc_mid_body,
    grid=(N_NODES // BLK,),
    in_specs=[_h16_spec(), _sum_spec(0), _sum_spec(1),
              pl.BlockSpec((BLK, 1), lambda i: (i, 0))],
    out_specs=_h16_spec(),
    out_shape=jax.ShapeDtypeStruct((2 * N_NODES, 16), jnp.float32),
)

_tc_out = pl.pallas_call(
    _tc_out_body,
    grid=(N_NODES // BLK,),
    in_specs=[_h16_spec(), _sum_spec(0), _sum_spec(1),
              pl.BlockSpec((BLK, 1), lambda i: (i, 0)),
              _full_spec((32, 32)), _full_spec((1, 32))],
    out_specs=pl.BlockSpec((BLK, 32), lambda i: (i, 0)),
    out_shape=jax.ShapeDtypeStruct((N_NODES, 32), jnp.float32),
)


def _prep_edges(edge_index):
    src = edge_index[0].astype(jnp.int32)
    dst = edge_index[1].astype(jnp.int32)
    pad = E_PAD - N_EDGES
    src = jnp.concatenate([src, jnp.zeros((pad,), jnp.int32)])
    dst = jnp.concatenate([dst, jnp.full((pad,), N_NODES, jnp.int32)])
    return jnp.stack([src.reshape(NS, EB, EJ, 128),
                      dst.reshape(NS, EB, EJ, 128)], axis=2)


def _mean_inputs(h16, edges):
    zacc = jnp.zeros((ACC_ROWS, 16), jnp.float32)
    zdeg = jnp.zeros((ACC_ROWS,), jnp.float32)
    sums, deg = _seg_call(h16, edges, zacc, zdeg)
    d = (deg[0, :N_NODES] + deg[1, :N_NODES]).reshape(N_NODES, 1)
    return sums, d


@jax.jit
def kernel(x, edge_index1, edge_index2, emb0, emb1, emb2, emb3,
           W_in, b_in, W_out, b_out):
    tbl = jnp.concatenate([emb0, emb1, emb2, emb3], axis=0)
    offs = jnp.array([0, 1000, 2000, 2100], jnp.int32)
    gidx = (x.astype(jnp.int32).T + offs[:, None]).reshape(-1)
    gidx = jnp.concatenate(
        [gidx, jnp.zeros((GATHER_ROWS_PAD - 4 * N_NODES,), jnp.int32)])
    feats_fm = _gather_call(tbl, gidx.reshape(NC * NS, GKB, 128))

    h16 = _tc_in(feats_fm, feats_fm, feats_fm, feats_fm,
                 W_in, b_in.reshape(1, 32))

    e1 = _prep_edges(edge_index1)
    sums, d = _mean_inputs(h16, e1)
    h16 = _tc_mid(h16, sums, sums, d)

    e2 = _prep_edges(edge_index2)
    sums, d = _mean_inputs(h16, e2)
    return _tc_out(h16, sums, sums, d, W_out, b_out.reshape(1, 32))


# speedup vs baseline: 11.1088x; 1.1102x over previous
import jax
import jax.numpy as jnp
from jax import lax
from jax.experimental import pallas as pl
from jax.experimental.pallas import tpu as pltpu
from jax.experimental.pallas import tpu_sc as plsc

N_NODES = 100000
N_EDGES = 1600000

NC = 2
NS = 16
L = 16

GATHER_ROWS_PAD = 409600
GKB = GATHER_ROWS_PAD // (NC * NS * 128)
GG = 4


def _gather_body(tbl_hbm, idx_hbm, out_hbm, idx_v, d0, d1, d2, d3,
                 gsem, osem):
    c = lax.axis_index("c")
    s = lax.axis_index("s")
    wid = s * NC + c
    data = (d0, d1, d2, d3)
    pltpu.sync_copy(idx_hbm.at[wid], idx_v)

    def out_ref(g, j):
        base = wid * (GKB * 128) + (g * GG + j) * 128
        return out_hbm.at[pl.ds(base, 128)]

    def group(g, carry):
        @pl.when(g > 0)
        def _():
            for j in range(GG):
                pltpu.make_async_copy(data[j], out_ref(g - 1, j),
                                      osem).wait()
        descs = [
            pltpu.async_copy(tbl_hbm.at[idx_v.at[g * GG + j]], data[j],
                             gsem)
            for j in range(GG)]
        for j in range(GG):
            descs[j].wait()
        for j in range(GG):
            pltpu.async_copy(data[j], out_ref(g, j), osem)
        return carry

    lax.fori_loop(0, GKB // GG, group, 0)
    for j in range(GG):
        pltpu.make_async_copy(data[j], out_ref(GKB // GG - 1, j),
                              osem).wait()


_gather_call = pl.kernel(
    _gather_body,
    out_type=jax.ShapeDtypeStruct((GATHER_ROWS_PAD, 16), jnp.float32),
    compiler_params=pltpu.CompilerParams(use_tc_tiling_on_sc=False),
    mesh=plsc.VectorSubcoreMesh(
        core_axis_name="c", subcore_axis_name="s", num_cores=NC,
        num_subcores=NS),
    scratch_types=[
        pltpu.VMEM((GKB, 128), jnp.int32),
        pltpu.VMEM((128, 16), jnp.float32),
        pltpu.VMEM((128, 16), jnp.float32),
        pltpu.VMEM((128, 16), jnp.float32),
        pltpu.VMEM((128, 16), jnp.float32),
        pltpu.SemaphoreType.DMA,
        pltpu.SemaphoreType.DMA,
    ],
)

EB = 196
EJ = 4
E_PAD = NS * EB * EJ * 128
ACC_ROWS = 100096
ACC_SLICE = ACC_ROWS // NS


def _seg_body(h0_hbm, h1_hbm, idx_hbm, zacc_hbm, zdeg_hbm,
              sums_hbm, deg_hbm,
              ia, ib, a0, a1, a2, a3, b0, b1, b2, b3, ones_v,
              acc_sh, deg_sh, gsem, ssa, ssb):
    c = lax.axis_index("c")
    s = lax.axis_index("s")
    slots = ((ia, (a0, a1, a2, a3), ssa),
             (ib, (b0, b1, b2, b3), ssb))

    pltpu.sync_copy(zacc_hbm.at[pl.ds(s * ACC_SLICE, ACC_SLICE)],
                    acc_sh.at[pl.ds(s * ACC_SLICE, ACC_SLICE)])
    pltpu.sync_copy(zdeg_hbm.at[pl.ds(s * ACC_SLICE, ACC_SLICE)],
                    deg_sh.at[pl.ds(s * ACC_SLICE, ACC_SLICE)])
    for t in range(128 // L):
        ones_v[pl.ds(t * L, L)] = jnp.ones((L,), jnp.float32)
    plsc.subcore_barrier()

    def fire_scatters(idx_v, data, ssem):
        for j in range(EJ):
            pltpu.async_copy(data[j], acc_sh.at[idx_v.at[1, j]], ssem,
                             add=True)
        @pl.when(c == 0)
        def _():
            for j in range(EJ // 2):
                pltpu.async_copy(ones_v, deg_sh.at[idx_v.at[1, j]], ssem,
                                 add=True)

        @pl.when(c == 1)
        def _():
            for j in range(EJ // 2, EJ):
                pltpu.async_copy(ones_v, deg_sh.at[idx_v.at[1, j]], ssem,
                                 add=True)

    def drain_scatters(idx_v, data, ssem):
        for j in range(EJ):
            pltpu.make_async_copy(data[j], acc_sh.at[idx_v.at[1, j]],
                                  ssem).wait()

        @pl.when(c == 0)
        def _():
            for j in range(EJ // 2):
                pltpu.make_async_copy(ones_v, deg_sh.at[idx_v.at[1, j]],
                                      ssem).wait()

        @pl.when(c == 1)
        def _():
            for j in range(EJ // 2, EJ):
                pltpu.make_async_copy(ones_v, deg_sh.at[idx_v.at[1, j]],
                                      ssem).wait()

    def half_step(tbl_hbm, i, slot_id, block):
        idx_v, data, ssem = slots[slot_id]
        @pl.when(i > 0)
        def _():
            drain_scatters(idx_v, data, ssem)
        pltpu.sync_copy(idx_hbm.at[s, block], idx_v)
        descs = [
            pltpu.async_copy(tbl_hbm.at[idx_v.at[0, j]], data[j], gsem)
            for j in range(EJ)]
        for j in range(EJ):
            descs[j].wait()
        fire_scatters(idx_v, data, ssem)

    def pipeline(tbl_hbm):
        def pair(i, carry):
            half_step(tbl_hbm, i, 0, 2 * i)
            half_step(tbl_hbm, i, 1, 2 * i + 1)
            return carry

        lax.fori_loop(0, EB // 2, pair, 0)

    @pl.when(c == 0)
    def _():
        pipeline(h0_hbm)

    @pl.when(c == 1)
    def _():
        pipeline(h1_hbm)

    for idx_v, data, ssem in slots:
        drain_scatters(idx_v, data, ssem)
    plsc.subcore_barrier()

    pltpu.sync_copy(acc_sh.at[pl.ds(s * ACC_SLICE, ACC_SLICE)],
                    sums_hbm.at[c, pl.ds(s * ACC_SLICE, ACC_SLICE)])
    pltpu.sync_copy(deg_sh.at[pl.ds(s * ACC_SLICE, ACC_SLICE)],
                    deg_hbm.at[c, pl.ds(s * ACC_SLICE, ACC_SLICE)])


_seg_call = pl.kernel(
    _seg_body,
    out_type=(
        jax.ShapeDtypeStruct((NC, ACC_ROWS, 16), jnp.float32),
        jax.ShapeDtypeStruct((NC, ACC_ROWS), jnp.float32),
    ),
    compiler_params=pltpu.CompilerParams(use_tc_tiling_on_sc=False),
    mesh=plsc.VectorSubcoreMesh(
        core_axis_name="c", subcore_axis_name="s", num_cores=NC,
        num_subcores=NS),
    scratch_types=[
        pltpu.VMEM((2, EJ, 128), jnp.int32),
        pltpu.VMEM((2, EJ, 128), jnp.int32),
        pltpu.VMEM((128, 16), jnp.float32),
        pltpu.VMEM((128, 16), jnp.float32),
        pltpu.VMEM((128, 16), jnp.float32),
        pltpu.VMEM((128, 16), jnp.float32),
        pltpu.VMEM((128, 16), jnp.float32),
        pltpu.VMEM((128, 16), jnp.float32),
        pltpu.VMEM((128, 16), jnp.float32),
        pltpu.VMEM((128, 16), jnp.float32),
        pltpu.VMEM((128,), jnp.float32),
        pltpu.VMEM_SHARED((ACC_ROWS, 16), jnp.float32),
        pltpu.VMEM_SHARED((ACC_ROWS,), jnp.float32),
        pltpu.SemaphoreType.DMA,
        pltpu.SemaphoreType.DMA,
        pltpu.SemaphoreType.DMA,
    ],
)


BLK = 2000


def _tc_in_body(x0_ref, x1_ref, x2_ref, x3_ref, w_ref, b_ref,
                o0_ref, o1_ref):
    xs = (x0_ref, x1_ref, x2_ref, x3_ref)
    h = b_ref[...]
    for f in range(4):
        h = h + jnp.dot(xs[f][...], w_ref[f * 16:(f + 1) * 16, :],
                        preferred_element_type=jnp.float32)
    o0_ref[...] = h[:, :16]
    o1_ref[...] = h[:, 16:]


def _tc_mid_body(h0_ref, h1_ref, s0_ref, s1_ref, d_ref, o0_ref, o1_ref):
    r = 1.0 / jnp.maximum(d_ref[...], 1.0)
    o0_ref[...] = jnp.maximum(h0_ref[...] + s0_ref[0] * r, 0.0)
    o1_ref[...] = jnp.maximum(h1_ref[...] + s1_ref[0] * r, 0.0)


def _tc_out_body(h0_ref, h1_ref, s0_ref, s1_ref, d_ref, w_ref, b_ref,
                 o_ref):
    r = 1.0 / jnp.maximum(d_ref[...], 1.0)
    h = jnp.concatenate([h0_ref[...] + s0_ref[0] * r,
                         h1_ref[...] + s1_ref[0] * r], axis=1)
    o_ref[...] = (
        jnp.dot(h, w_ref[...], preferred_element_type=jnp.float32)
        + b_ref[...])


def _full_spec(shape):
    return pl.BlockSpec(shape, lambda i: tuple(0 for _ in shape))


def _half_spec():
    return pl.BlockSpec((BLK, 16), lambda i: (i, 0))


def _sum_spec(core):
    return pl.BlockSpec((1, BLK, 16), lambda i, core=core: (core, i, 0))


_h_half = jax.ShapeDtypeStruct((N_NODES, 16), jnp.float32)

_tc_in = pl.pallas_call(
    _tc_in_body,
    grid=(N_NODES // BLK,),
    in_specs=[pl.BlockSpec((BLK, 16), lambda i, f=f: (f * (N_NODES // BLK)
                                                      + i, 0))
              for f in range(4)]
    + [_full_spec((64, 32)), _full_spec((1, 32))],
    out_specs=(_half_spec(), _half_spec()),
    out_shape=(_h_half, _h_half),
)

_tc_mid = pl.pallas_call(
    _tc_mid_body,
    grid=(N_NODES // BLK,),
    in_specs=[_half_spec(), _half_spec(), _sum_spec(0), _sum_spec(1),
              pl.BlockSpec((BLK, 1), lambda i: (i, 0))],
    out_specs=(_half_spec(), _half_spec()),
    out_shape=(_h_half, _h_half),
)

_tc_out = pl.pallas_call(
    _tc_out_body,
    grid=(N_NODES // BLK,),
    in_specs=[_half_spec(), _half_spec(), _sum_spec(0), _sum_spec(1),
              pl.BlockSpec((BLK, 1), lambda i: (i, 0)),
              _full_spec((32, 32)), _full_spec((1, 32))],
    out_specs=pl.BlockSpec((BLK, 32), lambda i: (i, 0)),
    out_shape=jax.ShapeDtypeStruct((N_NODES, 32), jnp.float32),
)


def _prep_edges(edge_index):
    src = edge_index[0].astype(jnp.int32)
    dst = edge_index[1].astype(jnp.int32)
    pad = E_PAD - N_EDGES
    src = jnp.concatenate([src, jnp.zeros((pad,), jnp.int32)])
    dst = jnp.concatenate([dst, jnp.full((pad,), N_NODES, jnp.int32)])
    return jnp.stack([src.reshape(NS, EB, EJ, 128),
                      dst.reshape(NS, EB, EJ, 128)], axis=2)


def _mean_inputs(h0, h1, edges):
    zacc = jnp.zeros((ACC_ROWS, 16), jnp.float32)
    zdeg = jnp.zeros((ACC_ROWS,), jnp.float32)
    sums, deg = _seg_call(h0, h1, edges, zacc, zdeg)
    d = (deg[0, :N_NODES] + deg[1, :N_NODES]).reshape(N_NODES, 1)
    return sums, d


@jax.jit
def kernel(x, edge_index1, edge_index2, emb0, emb1, emb2, emb3,
           W_in, b_in, W_out, b_out):
    tbl = jnp.concatenate([emb0, emb1, emb2, emb3], axis=0)
    offs = jnp.array([0, 1000, 2000, 2100], jnp.int32)
    gidx = (x.astype(jnp.int32).T + offs[:, None]).reshape(-1)
    gidx = jnp.concatenate(
        [gidx, jnp.zeros((GATHER_ROWS_PAD - 4 * N_NODES,), jnp.int32)])
    feats_fm = _gather_call(tbl, gidx.reshape(NC * NS, GKB, 128))

    h0, h1 = _tc_in(feats_fm, feats_fm, feats_fm, feats_fm,
                    W_in, b_in.reshape(1, 32))

    e1 = _prep_edges(edge_index1)
    sums, d = _mean_inputs(h0, h1, e1)
    h0, h1 = _tc_mid(h0, h1, sums, sums, d)

    e2 = _prep_edges(edge_index2)
    sums, d = _mean_inputs(h0, h1, e2)
    return _tc_out(h0, h1, sums, sums, d, W_out, b_out.reshape(1, 32))
